# Initial kernel scaffold; baseline (speedup 1.0000x reference)
import jax
import jax.numpy as jnp
from jax.experimental import pallas as pl


def kernel(inp, table):
    raise NotImplementedError("write your pallas kernel here")


# speedup vs baseline: 1.4618x; 1.4618x over previous
import jax
import jax.numpy as jnp
from jax import lax
from jax.experimental import pallas as pl
from jax.experimental.pallas import tpu as pltpu
from jax.experimental.pallas import tpu_sc as plsc

_DIM = 32
_NC, _NS = 2, 16
_NW = _NC * _NS
_G = 128
_K = 8
_CHUNK = _K * _G


def _emb_body(idx_hbm, table_hbm, out_hbm, idx_v, rows_v, sem):
    n_groups = idx_hbm.shape[0]
    g_per_w = n_groups // _NW
    n_chunks = g_per_w // _K
    wid = lax.axis_index("s") * _NC + lax.axis_index("c")
    base = wid * g_per_w

    @pl.loop(0, n_chunks)
    def _chunk(c):
        off = base + c * _K
        pltpu.sync_copy(idx_hbm.at[pl.ds(off, _K)], idx_v)
        copies = [
            pltpu.async_copy(table_hbm.at[idx_v.at[j]], rows_v.at[j], sem)
            for j in range(_K)
        ]
        for cp in copies:
            cp.wait()
        pltpu.sync_copy(rows_v, out_hbm.at[pl.ds(off, _K)])


def kernel(inp, table):
    b, l = inp.shape
    n = b * l
    idx = inp.reshape(n // _G, _G).astype(jnp.int32)
    mesh = plsc.VectorSubcoreMesh(core_axis_name="c", subcore_axis_name="s")
    out = pl.kernel(
        _emb_body,
        out_type=jax.ShapeDtypeStruct((n // _G, _G, _DIM), table.dtype),
        mesh=mesh,
        scratch_types=[
            pltpu.VMEM((_K, _G), jnp.int32),
            pltpu.VMEM((_K, _G, _DIM), jnp.float32),
            pltpu.SemaphoreType.DMA,
        ],
        compiler_params=pltpu.CompilerParams(use_tc_tiling_on_sc=False),
    )(idx, table)
    return out.reshape(b, l, _DIM)


# speedup vs baseline: 1.4955x; 1.0231x over previous
import jax
import jax.numpy as jnp
from jax import lax
from jax.experimental import pallas as pl
from jax.experimental.pallas import tpu as pltpu
from jax.experimental.pallas import tpu_sc as plsc

_DIM = 32
_NC, _NS = 2, 16
_NW = _NC * _NS
_G = 128
_K = 8


def _emb_body(idx_hbm, table_hbm, out_hbm, idx_v, rows_v, semi0, semi1,
              semg0, semg1, semo0, semo1):
    n_groups = idx_hbm.shape[0]
    g_per_w = n_groups // _NW
    n_chunks = g_per_w // _K
    wid = lax.axis_index("s") * _NC + lax.axis_index("c")
    base = wid * g_per_w

    semi = (semi0, semi1)
    semg = (semg0, semg1)
    semo = (semo0, semo1)

    def idx_copy(c, s):
        return pltpu.make_async_copy(
            idx_hbm.at[pl.ds(base + c * _K, _K)], idx_v.at[s], semi[s])

    def fire_gathers(s):
        for j in range(_K):
            pltpu.async_copy(table_hbm.at[idx_v.at[s].at[j]],
                             rows_v.at[s].at[j], semg[s])

    def drain_gathers(s):
        pltpu.make_async_copy(out_hbm.at[pl.ds(0, _K)], rows_v.at[s],
                              semg[s]).wait()

    def out_copy(c, s):
        return pltpu.make_async_copy(
            rows_v.at[s], out_hbm.at[pl.ds(base + c * _K, _K)], semo[s])

    idx_copy(0, 0).start()
    idx_copy(1, 1).start()
    idx_copy(0, 0).wait()
    fire_gathers(0)

    drain_gathers(0)
    idx_copy(2, 0).start()
    idx_copy(1, 1).wait()
    fire_gathers(1)
    out_copy(0, 0).start()

    drain_gathers(1)
    idx_copy(3, 1).start()
    out_copy(0, 0).wait()
    idx_copy(2, 0).wait()
    fire_gathers(0)
    out_copy(1, 1).start()

    @pl.loop(0, (n_chunks - 2) // 2)
    def _pair(t):
        for b in range(2):
            c = 2 + 2 * t + b
            s = b
            drain_gathers(s)

            @pl.when(c + 2 < n_chunks)
            def _():
                idx_copy(c + 2, s).start()

            out_copy(c - 1, 1 - s).wait()
            idx_copy(c + 1, 1 - s).wait()
            fire_gathers(1 - s)
            out_copy(c, s).start()

    c = n_chunks - 1
    drain_gathers(0)
    out_copy(c - 1, 1).wait()
    out_copy(c, 0).start()
    out_copy(c, 0).wait()


def kernel(inp, table):
    b, l = inp.shape
    n = b * l
    idx = inp.reshape(n // _G, _G).astype(jnp.int32)
    mesh = plsc.VectorSubcoreMesh(core_axis_name="c", subcore_axis_name="s")
    out = pl.kernel(
        _emb_body,
        out_type=jax.ShapeDtypeStruct((n // _G, _G, _DIM), table.dtype),
        mesh=mesh,
        scratch_types=[
            pltpu.VMEM((2, _K, _G), jnp.int32),
            pltpu.VMEM((2, _K, _G, _DIM), jnp.float32),
            pltpu.SemaphoreType.DMA,
            pltpu.SemaphoreType.DMA,
            pltpu.SemaphoreType.DMA,
            pltpu.SemaphoreType.DMA,
            pltpu.SemaphoreType.DMA,
            pltpu.SemaphoreType.DMA,
        ],
        compiler_params=pltpu.CompilerParams(use_tc_tiling_on_sc=False),
    )(idx, table)
    return out.reshape(b, l, _DIM)


# speedup vs baseline: 1.4963x; 1.0005x over previous
import jax
import jax.numpy as jnp
from jax import lax
from jax.experimental import pallas as pl
from jax.experimental.pallas import tpu as pltpu
from jax.experimental.pallas import tpu_sc as plsc

_DIM = 32
_NC, _NS = 2, 16
_NW = _NC * _NS
_C = 1024


def _emb_body(idx_hbm, table_hbm, out_hbm, idx_v, rows_v, semi0, semi1,
              semg0, semg1, semo0, semo1):
    n_rows = idx_hbm.shape[0]
    r_per_w = n_rows // _NW
    n_chunks = r_per_w // _C
    wid = lax.axis_index("s") * _NC + lax.axis_index("c")
    base = wid * r_per_w

    semi = (semi0, semi1)
    semg = (semg0, semg1)
    semo = (semo0, semo1)

    def idx_copy(c, s):
        return pltpu.make_async_copy(
            idx_hbm.at[pl.ds(base + c * _C, _C)], idx_v.at[s], semi[s])

    def fire_gathers(s):
        pltpu.async_copy(table_hbm.at[idx_v.at[s]], rows_v.at[s], semg[s])

    def drain_gathers(s):
        pltpu.make_async_copy(out_hbm.at[pl.ds(0, _C)], rows_v.at[s],
                              semg[s]).wait()

    def out_copy(c, s):
        return pltpu.make_async_copy(
            rows_v.at[s], out_hbm.at[pl.ds(base + c * _C, _C)], semo[s])

    idx_copy(0, 0).start()
    idx_copy(1, 1).start()
    idx_copy(0, 0).wait()
    fire_gathers(0)

    drain_gathers(0)
    idx_copy(2, 0).start()
    idx_copy(1, 1).wait()
    fire_gathers(1)
    out_copy(0, 0).start()

    drain_gathers(1)
    idx_copy(3, 1).start()
    out_copy(0, 0).wait()
    idx_copy(2, 0).wait()
    fire_gathers(0)
    out_copy(1, 1).start()

    @pl.loop(0, (n_chunks - 2) // 2)
    def _pair(t):
        for b in range(2):
            c = 2 + 2 * t + b
            s = b
            drain_gathers(s)

            @pl.when(c + 2 < n_chunks)
            def _():
                idx_copy(c + 2, s).start()

            out_copy(c - 1, 1 - s).wait()
            idx_copy(c + 1, 1 - s).wait()
            fire_gathers(1 - s)
            out_copy(c, s).start()

    c = n_chunks - 1
    drain_gathers(0)
    out_copy(c - 1, 1).wait()
    out_copy(c, 0).start()
    out_copy(c, 0).wait()


def kernel(inp, table):
    b, l = inp.shape
    n = b * l
    idx = inp.reshape(n).astype(jnp.int32)
    mesh = plsc.VectorSubcoreMesh(core_axis_name="c", subcore_axis_name="s")
    out = pl.kernel(
        _emb_body,
        out_type=jax.ShapeDtypeStruct((n, _DIM), table.dtype),
        mesh=mesh,
        scratch_types=[
            pltpu.VMEM((2, _C), jnp.int32),
            pltpu.VMEM((2, _C, _DIM), jnp.float32),
            pltpu.SemaphoreType.DMA,
            pltpu.SemaphoreType.DMA,
            pltpu.SemaphoreType.DMA,
            pltpu.SemaphoreType.DMA,
            pltpu.SemaphoreType.DMA,
            pltpu.SemaphoreType.DMA,
        ],
        compiler_params=pltpu.CompilerParams(use_tc_tiling_on_sc=False),
    )(idx, table)
    return out.reshape(b, l, _DIM)
